# baseline (speedup 1.0000x reference)
import functools

import jax
import jax.numpy as jnp
from jax import lax
from jax.experimental import pallas as pl
from jax.experimental.pallas import tpu as pltpu
from jax.experimental.pallas import tpu_sc as plsc

NC = 2
NS = 16
LANES = 16
BATCH = 16384
B_TOTAL = 2 * BATCH
NW = NC * NS
B_PER_W = B_TOTAL // NW
CHUNK = 128
NCHUNK = B_PER_W // CHUNK
DIM = 64
DGROUPS = DIM // LANES

_mesh = plsc.VectorSubcoreMesh(
    core_axis_name="c", subcore_axis_name="s", num_cores=NC, num_subcores=NS)


@functools.partial(
    pl.kernel,
    out_type=jax.ShapeDtypeStruct((B_TOTAL,), jnp.float32),
    mesh=_mesh,
    compiler_params=pltpu.CompilerParams(
        needs_layout_passes=False, use_tc_tiling_on_sc=False),
    scratch_types=[
        pltpu.VMEM((CHUNK,), jnp.int32),
        pltpu.VMEM((CHUNK,), jnp.int32),
        pltpu.VMEM((CHUNK,), jnp.int32),
        pltpu.VMEM((CHUNK, DIM), jnp.float32),
        pltpu.VMEM((CHUNK, DIM), jnp.float32),
        pltpu.VMEM((CHUNK, DIM), jnp.float32),
        pltpu.VMEM((LANES * CHUNK,), jnp.float32),
        pltpu.VMEM((CHUNK,), jnp.float32),
        pltpu.SemaphoreType.DMA,
    ],
)
def _transe_sc(ent_hbm, rel_hbm, hidx_hbm, ridx_hbm, tidx_hbm, out_hbm,
               hidx_v, ridx_v, tidx_v, hrows_v, rrows_v, trows_v,
               part_v, outbuf_v, sem):
    wid = lax.axis_index("s") * NC + lax.axis_index("c")
    base = wid * B_PER_W
    lane_iota = lax.iota(jnp.int32, LANES)

    for k in range(NCHUNK):
        cb = base + k * CHUNK
        pltpu.sync_copy(hidx_hbm.at[pl.ds(cb, CHUNK)], hidx_v)
        pltpu.sync_copy(ridx_hbm.at[pl.ds(cb, CHUNK)], ridx_v)
        pltpu.sync_copy(tidx_hbm.at[pl.ds(cb, CHUNK)], tidx_v)
        ch = pltpu.async_copy(ent_hbm.at[hidx_v], hrows_v, sem)
        cr = pltpu.async_copy(rel_hbm.at[ridx_v], rrows_v, sem)
        ct = pltpu.async_copy(ent_hbm.at[tidx_v], trows_v, sem)
        ch.wait()
        cr.wait()
        ct.wait()

        def row_body(i, carry):
            acc = jnp.zeros((LANES,), jnp.float32)
            for c in range(DGROUPS):
                sl = pl.ds(c * LANES, LANES)
                x = hrows_v[i, sl] + rrows_v[i, sl] - trows_v[i, sl]
                u = jnp.exp(x)
                n = u * (u + 2.0)
                y = x * (n / (n + 2.0))
                acc = acc + y * y
            plsc.store_scatter(part_v, [lane_iota * CHUNK + i], acc)
            return carry

        lax.fori_loop(0, CHUNK, row_body, 0)

        for j in range(CHUNK // LANES):
            sl = pl.ds(j * LANES, LANES)
            s = part_v[pl.ds(j * LANES, LANES)]
            for l in range(1, LANES):
                s = s + part_v[pl.ds(l * CHUNK + j * LANES, LANES)]
            seed = lax.shift_right_logical(
                plsc.bitcast(s, jnp.int32), 1) + jnp.int32(0x1FBD1DF5)
            t = plsc.bitcast(seed, jnp.float32)
            t = 0.5 * (t + s / t)
            t = 0.5 * (t + s / t)
            outbuf_v[sl] = t

        pltpu.sync_copy(outbuf_v, out_hbm.at[pl.ds(cb, CHUNK)])


def kernel(positive_triplets, negative_triplets, offset, entities_emb, relations_emb):
    trip = jnp.concatenate([positive_triplets, negative_triplets], axis=0)
    hidx = trip[:, 0]
    ridx = trip[:, 1]
    tidx = trip[:, 2]
    dist = _transe_sc(entities_emb, relations_emb, hidx, ridx, tidx)
    return (dist[:BATCH], dist[BATCH:])
